# baseline (speedup 1.0000x reference)
import functools

import jax
import jax.numpy as jnp
from jax import lax
from jax.experimental import pallas as pl
from jax.experimental.pallas import tpu as pltpu
from jax.experimental.pallas import tpu_sc as plsc

_NUM_CLS = 80
_POS_T = 0.5
_NEG_T = 0.4

_N = 100
_NP = 112
_B = 8
_A = 20000
_WPI = 4
_CHUNK = 1024
_NCHUNK = 5
_STRIDE = 4960
_GROUPS = _CHUNK // 16


def _sc_body(ann_hbm, anct_hbm, cls_hbm, reg_hbm, st_hbm,
             bx1_v, by1_v, bx2_v, by2_v, bar_v, bcl_v,
             ax1_v, ay1_v, ax2_v, ay2_v, cls_v, reg_v, st_v):
    wid = lax.axis_index("s") * 2 + lax.axis_index("c")
    b = wid // _WPI
    q = wid % _WPI

    ann0 = b * (6 * _NP)
    pltpu.sync_copy(ann_hbm.at[pl.ds(ann0 + 0 * _NP, _NP)], bx1_v)
    pltpu.sync_copy(ann_hbm.at[pl.ds(ann0 + 1 * _NP, _NP)], by1_v)
    pltpu.sync_copy(ann_hbm.at[pl.ds(ann0 + 2 * _NP, _NP)], bx2_v)
    pltpu.sync_copy(ann_hbm.at[pl.ds(ann0 + 3 * _NP, _NP)], by2_v)
    pltpu.sync_copy(ann_hbm.at[pl.ds(ann0 + 4 * _NP, _NP)], bar_v)
    pltpu.sync_copy(ann_hbm.at[pl.ds(ann0 + 5 * _NP, _NP)], bcl_v)

    iota = lax.iota(jnp.int32, 16)
    iota4 = iota * 4
    iota80 = iota * _NUM_CLS
    ones = jnp.ones((16,), jnp.float32)
    zeros16 = jnp.zeros((16,), jnp.float32)
    zeros16i = jnp.zeros((16,), jnp.int32)
    ones16i = jnp.ones((16,), jnp.int32)

    def chunk_body(c, carry):
        a0 = q * _STRIDE + c * _CHUNK
        pltpu.sync_copy(anct_hbm.at[pl.ds(0 * _A + a0, _CHUNK)], ax1_v)
        pltpu.sync_copy(anct_hbm.at[pl.ds(1 * _A + a0, _CHUNK)], ay1_v)
        pltpu.sync_copy(anct_hbm.at[pl.ds(2 * _A + a0, _CHUNK)], ax2_v)
        pltpu.sync_copy(anct_hbm.at[pl.ds(3 * _A + a0, _CHUNK)], ay2_v)

        def epilogue(g, ax1, ay1, ax2, ay2, aw, ah, bi, bS, bj):
            s = pl.ds(g * 16, 16)
            ua = bS - bi
            pos = bi >= _POS_T * ua
            ign = jnp.logical_and(bi > _NEG_T * ua, jnp.logical_not(pos))
            st_v[s] = jnp.where(pos, 1.0, jnp.where(ign, -1.0, 0.0))

            gx1 = plsc.load_gather(bx1_v, [bj])
            gy1 = plsc.load_gather(by1_v, [bj])
            gx2 = plsc.load_gather(bx2_v, [bj])
            gy2 = plsc.load_gather(by2_v, [bj])
            gcl = plsc.load_gather(bcl_v, [bj])

            s5w = 5.0 / aw
            s5h = 5.0 / ah
            i4 = iota4 + g * 64
            plsc.store_scatter(reg_v, [i4], (gx1 - ax1) * s5w)
            plsc.store_scatter(reg_v, [i4 + 1], (gy1 - ay1) * s5h)
            plsc.store_scatter(reg_v, [i4 + 2], (gx2 - ax2) * s5w)
            plsc.store_scatter(reg_v, [i4 + 3], (gy2 - ay2) * s5h)

            base = g * (16 * _NUM_CLS)
            for k in range(_NUM_CLS):
                cls_v[pl.ds(base + k * 16, 16)] = zeros16
            label = gcl.astype(jnp.int32)
            ci = iota80 + (base + label)
            plsc.store_scatter(cls_v, [ci], ones, mask=pos)

        def group_body(g2, carry2):
            gA = g2 * 2
            gB = gA + 1
            sA = pl.ds(gA * 16, 16)
            sB = pl.ds(gB * 16, 16)
            ax1A = ax1_v[sA]; ay1A = ay1_v[sA]; ax2A = ax2_v[sA]; ay2A = ay2_v[sA]
            ax1B = ax1_v[sB]; ay1B = ay1_v[sB]; ax2B = ax2_v[sB]; ay2B = ay2_v[sB]
            awA = ax2A - ax1A; ahA = ay2A - ay1A; aareaA = awA * ahA
            awB = ax2B - ax1B; ahB = ay2B - ay1B; aareaB = awB * ahB

            def box_body(j, st8):
                biA, bSA, bjA, biB, bSB, bjB, jv = st8
                bx1 = plsc.load_gather(bx1_v, [jv])
                by1 = plsc.load_gather(by1_v, [jv])
                bx2 = plsc.load_gather(bx2_v, [jv])
                by2 = plsc.load_gather(by2_v, [jv])
                bar = plsc.load_gather(bar_v, [jv])
                iwA = jnp.maximum(jnp.minimum(ax2A, bx2) - jnp.maximum(ax1A, bx1), 0.0)
                ihA = jnp.maximum(jnp.minimum(ay2A, by2) - jnp.maximum(ay1A, by1), 0.0)
                iwB = jnp.maximum(jnp.minimum(ax2B, bx2) - jnp.maximum(ax1B, bx1), 0.0)
                ihB = jnp.maximum(jnp.minimum(ay2B, by2) - jnp.maximum(ay1B, by1), 0.0)
                interA = iwA * ihA
                interB = iwB * ihB
                SA = aareaA + bar
                SB = aareaB + bar
                mA = interA * bSA > biA * SA
                mB = interB * bSB > biB * SB
                biA = jnp.where(mA, interA, biA)
                bSA = jnp.where(mA, SA, bSA)
                bjA = jnp.where(mA, jv, bjA)
                biB = jnp.where(mB, interB, biB)
                bSB = jnp.where(mB, SB, bSB)
                bjB = jnp.where(mB, jv, bjB)
                return (biA, bSA, bjA, biB, bSB, bjB, jv + ones16i)

            biA, bSA, bjA, biB, bSB, bjB, _jv = lax.fori_loop(
                0, _N, box_body,
                (zeros16, ones, zeros16i, zeros16, ones, zeros16i, zeros16i),
                unroll=4)

            epilogue(gA, ax1A, ay1A, ax2A, ay2A, awA, ahA, biA, bSA, bjA)
            epilogue(gB, ax1B, ay1B, ax2B, ay2B, awB, ahB, biB, bSB, bjB)
            return carry2

        lax.fori_loop(0, _GROUPS // 2, group_body, None)

        row0 = b * _A + a0
        pltpu.sync_copy(cls_v, cls_hbm.at[pl.ds(row0 * _NUM_CLS, _CHUNK * _NUM_CLS)])
        pltpu.sync_copy(reg_v, reg_hbm.at[pl.ds(row0 * 4, _CHUNK * 4)])
        pltpu.sync_copy(st_v, st_hbm.at[pl.ds(row0, _CHUNK)])
        return carry

    lax.fori_loop(0, _NCHUNK, chunk_body, None)


def _sc_call(ann_flat, anct_flat):
    mesh = plsc.VectorSubcoreMesh(core_axis_name="c", subcore_axis_name="s")
    f = functools.partial(
        pl.kernel,
        out_type=[
            jax.ShapeDtypeStruct((_B * _A * _NUM_CLS,), jnp.float32),
            jax.ShapeDtypeStruct((_B * _A * 4,), jnp.float32),
            jax.ShapeDtypeStruct((_B * _A,), jnp.float32),
        ],
        mesh=mesh,
        compiler_params=pltpu.CompilerParams(needs_layout_passes=False),
        scratch_types=[
            pltpu.VMEM((_NP,), jnp.float32),
            pltpu.VMEM((_NP,), jnp.float32),
            pltpu.VMEM((_NP,), jnp.float32),
            pltpu.VMEM((_NP,), jnp.float32),
            pltpu.VMEM((_NP,), jnp.float32),
            pltpu.VMEM((_NP,), jnp.float32),
            pltpu.VMEM((_CHUNK,), jnp.float32),
            pltpu.VMEM((_CHUNK,), jnp.float32),
            pltpu.VMEM((_CHUNK,), jnp.float32),
            pltpu.VMEM((_CHUNK,), jnp.float32),
            pltpu.VMEM((_CHUNK * _NUM_CLS,), jnp.float32),
            pltpu.VMEM((_CHUNK * 4,), jnp.float32),
            pltpu.VMEM((_CHUNK,), jnp.float32),
        ],
    )(_sc_body)
    return f(ann_flat, anct_flat)


def kernel(annotations_batch, anchors):
    B, N, _ = annotations_batch.shape
    A = anchors.shape[0]
    assert (B, N, A) == (_B, _N, _A)
    boxes = annotations_batch[:, :, :4]
    barea = ((boxes[:, :, 2] - boxes[:, :, 0])
             * (boxes[:, :, 3] - boxes[:, :, 1]))[:, :, None]
    ann_rows = jnp.concatenate(
        [boxes, barea, annotations_batch[:, :, 4:5]], axis=-1)
    ann_t = jnp.transpose(ann_rows, (0, 2, 1))
    ann_t = jnp.pad(ann_t, ((0, 0), (0, 0), (0, _NP - _N)))
    ann_flat = ann_t.reshape(-1)
    anct_flat = jnp.transpose(anchors, (1, 0)).reshape(-1)
    cls, reg, st = _sc_call(ann_flat, anct_flat)
    return (cls.reshape(B, A, _NUM_CLS), reg.reshape(B, A, 4), st.reshape(B, A))


# speedup vs baseline: 1.0004x; 1.0004x over previous
import functools

import jax
import jax.numpy as jnp
from jax import lax
from jax.experimental import pallas as pl
from jax.experimental.pallas import tpu as pltpu
from jax.experimental.pallas import tpu_sc as plsc

_NUM_CLS = 80
_POS_T = 0.5
_NEG_T = 0.4

_N = 100
_NP = 112
_B = 8
_A = 20000
_WPI = 4
_CHUNK = 512
_NCHUNK = 10
_STRIDE = 4960
_GROUPS = _CHUNK // 16


def _sc_body(ann_hbm, anct_hbm, cls_hbm, reg_hbm, st_hbm,
             bx1_v, by1_v, bx2_v, by2_v, bar_v, bcl_v,
             ax1_v, ay1_v, ax2_v, ay2_v, cls_v, reg_v, st_v):
    wid = lax.axis_index("s") * 2 + lax.axis_index("c")
    b = wid // _WPI
    q = wid % _WPI

    ann0 = pl.multiple_of(b * (6 * _NP), 16)
    pltpu.sync_copy(ann_hbm.at[pl.ds(ann0 + 0 * _NP, _NP)], bx1_v)
    pltpu.sync_copy(ann_hbm.at[pl.ds(ann0 + 1 * _NP, _NP)], by1_v)
    pltpu.sync_copy(ann_hbm.at[pl.ds(ann0 + 2 * _NP, _NP)], bx2_v)
    pltpu.sync_copy(ann_hbm.at[pl.ds(ann0 + 3 * _NP, _NP)], by2_v)
    pltpu.sync_copy(ann_hbm.at[pl.ds(ann0 + 4 * _NP, _NP)], bar_v)
    pltpu.sync_copy(ann_hbm.at[pl.ds(ann0 + 5 * _NP, _NP)], bcl_v)

    iota = lax.iota(jnp.int32, 16)
    iota4 = iota * 4
    iota80 = iota * _NUM_CLS
    ones = jnp.ones((16,), jnp.float32)
    zeros16 = jnp.zeros((16,), jnp.float32)
    zeros16i = jnp.zeros((16,), jnp.int32)
    ones16i = jnp.ones((16,), jnp.int32)

    def chunk_body(c, carry):
        a0 = q * _STRIDE + c * _CHUNK
        a0 = pl.multiple_of(a0, 16)
        pltpu.sync_copy(anct_hbm.at[pl.ds(pl.multiple_of(0 * _A + a0, 16), _CHUNK)], ax1_v)
        pltpu.sync_copy(anct_hbm.at[pl.ds(pl.multiple_of(1 * _A + a0, 16), _CHUNK)], ay1_v)
        pltpu.sync_copy(anct_hbm.at[pl.ds(pl.multiple_of(2 * _A + a0, 16), _CHUNK)], ax2_v)
        pltpu.sync_copy(anct_hbm.at[pl.ds(pl.multiple_of(3 * _A + a0, 16), _CHUNK)], ay2_v)

        def epilogue(g, ax1, ay1, ax2, ay2, aw, ah, bi, bS, bj):
            s = pl.ds(g * 16, 16)
            ua = bS - bi
            pos = bi >= _POS_T * ua
            ign = jnp.logical_and(bi > _NEG_T * ua, jnp.logical_not(pos))
            st_v[s] = jnp.where(pos, 1.0, jnp.where(ign, -1.0, 0.0))

            gx1 = plsc.load_gather(bx1_v, [bj])
            gy1 = plsc.load_gather(by1_v, [bj])
            gx2 = plsc.load_gather(bx2_v, [bj])
            gy2 = plsc.load_gather(by2_v, [bj])
            gcl = plsc.load_gather(bcl_v, [bj])

            s5w = 5.0 / aw
            s5h = 5.0 / ah
            rows = iota + g * 16
            i4 = iota4 + g * 64
            plsc.store_scatter(reg_v, [i4], (gx1 - ax1) * s5w)
            plsc.store_scatter(reg_v, [i4 + 1], (gy1 - ay1) * s5h)
            plsc.store_scatter(reg_v, [i4 + 2], (gx2 - ax2) * s5w)
            plsc.store_scatter(reg_v, [i4 + 3], (gy2 - ay2) * s5h)

            for r in range(16):
                row = g * 16 + r
                for k in range(_NUM_CLS // 16):
                    cls_v[row, pl.ds(k * 16, 16)] = zeros16
            label = gcl.astype(jnp.int32)
            plsc.store_scatter(cls_v, [rows, label], ones, mask=pos)

        def group_body(g2, carry2):
            gA = g2 * 2
            gB = gA + 1
            sA = pl.ds(gA * 16, 16)
            sB = pl.ds(gB * 16, 16)
            ax1A = ax1_v[sA]; ay1A = ay1_v[sA]; ax2A = ax2_v[sA]; ay2A = ay2_v[sA]
            ax1B = ax1_v[sB]; ay1B = ay1_v[sB]; ax2B = ax2_v[sB]; ay2B = ay2_v[sB]
            awA = ax2A - ax1A; ahA = ay2A - ay1A; aareaA = awA * ahA
            awB = ax2B - ax1B; ahB = ay2B - ay1B; aareaB = awB * ahB

            def box_body(j, st8):
                biA, bSA, bjA, biB, bSB, bjB, jv = st8
                bx1 = plsc.load_gather(bx1_v, [jv])
                by1 = plsc.load_gather(by1_v, [jv])
                bx2 = plsc.load_gather(bx2_v, [jv])
                by2 = plsc.load_gather(by2_v, [jv])
                bar = plsc.load_gather(bar_v, [jv])
                iwA = jnp.maximum(jnp.minimum(ax2A, bx2) - jnp.maximum(ax1A, bx1), 0.0)
                ihA = jnp.maximum(jnp.minimum(ay2A, by2) - jnp.maximum(ay1A, by1), 0.0)
                iwB = jnp.maximum(jnp.minimum(ax2B, bx2) - jnp.maximum(ax1B, bx1), 0.0)
                ihB = jnp.maximum(jnp.minimum(ay2B, by2) - jnp.maximum(ay1B, by1), 0.0)
                interA = iwA * ihA
                interB = iwB * ihB
                SA = aareaA + bar
                SB = aareaB + bar
                mA = interA * bSA > biA * SA
                mB = interB * bSB > biB * SB
                biA = jnp.where(mA, interA, biA)
                bSA = jnp.where(mA, SA, bSA)
                bjA = jnp.where(mA, jv, bjA)
                biB = jnp.where(mB, interB, biB)
                bSB = jnp.where(mB, SB, bSB)
                bjB = jnp.where(mB, jv, bjB)
                return (biA, bSA, bjA, biB, bSB, bjB, jv + ones16i)

            biA, bSA, bjA, biB, bSB, bjB, _jv = lax.fori_loop(
                0, _N, box_body,
                (zeros16, ones, zeros16i, zeros16, ones, zeros16i, zeros16i),
                unroll=4)

            epilogue(gA, ax1A, ay1A, ax2A, ay2A, awA, ahA, biA, bSA, bjA)
            epilogue(gB, ax1B, ay1B, ax2B, ay2B, awB, ahB, biB, bSB, bjB)
            return carry2

        lax.fori_loop(0, _GROUPS // 2, group_body, None)

        a0w = pl.multiple_of(a0, 16)
        row0 = pl.multiple_of(b * _A + a0, 16)
        pltpu.sync_copy(cls_v, cls_hbm.at[b, pl.ds(a0w, _CHUNK), :])
        pltpu.sync_copy(reg_v, reg_hbm.at[pl.ds(pl.multiple_of(row0 * 4, 16), _CHUNK * 4)])
        pltpu.sync_copy(st_v, st_hbm.at[pl.ds(row0, _CHUNK)])
        return carry

    lax.fori_loop(0, _NCHUNK, chunk_body, None)


def _sc_call(ann_flat, anct_flat):
    mesh = plsc.VectorSubcoreMesh(core_axis_name="c", subcore_axis_name="s")
    f = functools.partial(
        pl.kernel,
        out_type=[
            jax.ShapeDtypeStruct((_B, _A, _NUM_CLS), jnp.float32),
            jax.ShapeDtypeStruct((_B * _A * 4,), jnp.float32),
            jax.ShapeDtypeStruct((_B * _A,), jnp.float32),
        ],
        mesh=mesh,
        compiler_params=pltpu.CompilerParams(needs_layout_passes=False),
        scratch_types=[
            pltpu.VMEM((_NP,), jnp.float32),
            pltpu.VMEM((_NP,), jnp.float32),
            pltpu.VMEM((_NP,), jnp.float32),
            pltpu.VMEM((_NP,), jnp.float32),
            pltpu.VMEM((_NP,), jnp.float32),
            pltpu.VMEM((_NP,), jnp.float32),
            pltpu.VMEM((_CHUNK,), jnp.float32),
            pltpu.VMEM((_CHUNK,), jnp.float32),
            pltpu.VMEM((_CHUNK,), jnp.float32),
            pltpu.VMEM((_CHUNK,), jnp.float32),
            pltpu.VMEM((_CHUNK, _NUM_CLS), jnp.float32),
            pltpu.VMEM((_CHUNK * 4,), jnp.float32),
            pltpu.VMEM((_CHUNK,), jnp.float32),
        ],
    )(_sc_body)
    return f(ann_flat, anct_flat)


def kernel(annotations_batch, anchors):
    B, N, _ = annotations_batch.shape
    A = anchors.shape[0]
    assert (B, N, A) == (_B, _N, _A)
    boxes = annotations_batch[:, :, :4]
    barea = ((boxes[:, :, 2] - boxes[:, :, 0])
             * (boxes[:, :, 3] - boxes[:, :, 1]))[:, :, None]
    ann_rows = jnp.concatenate(
        [boxes, barea, annotations_batch[:, :, 4:5]], axis=-1)
    ann_t = jnp.transpose(ann_rows, (0, 2, 1))
    ann_t = jnp.pad(ann_t, ((0, 0), (0, 0), (0, _NP - _N)))
    ann_flat = ann_t.reshape(-1)
    anct_flat = jnp.transpose(anchors, (1, 0)).reshape(-1)
    cls, reg, st = _sc_call(ann_flat, anct_flat)
    return (cls, reg.reshape(B, A, 4), st.reshape(B, A))


# speedup vs baseline: 1.0106x; 1.0101x over previous
import functools

import jax
import jax.numpy as jnp
from jax import lax
from jax.experimental import pallas as pl
from jax.experimental.pallas import tpu as pltpu
from jax.experimental.pallas import tpu_sc as plsc

_NUM_CLS = 80
_POS_T = 0.5
_NEG_T = 0.4

_N = 100
_NP = 112
_B = 8
_A = 20000
_WPI = 4
_CHUNK = 1024
_NCHUNK = 5
_STRIDE = 4960
_GROUPS = _CHUNK // 16


def _sc_body(ann_hbm, anct_hbm, cls_hbm, reg_hbm, st_hbm,
             bx1_v, by1_v, bx2_v, by2_v, bar_v, bcl_v,
             ax1_v, ay1_v, ax2_v, ay2_v, cls_v, reg_v, st_v):
    wid = lax.axis_index("s") * 2 + lax.axis_index("c")
    b = wid // _WPI
    q = wid % _WPI

    ann0 = b * (6 * _NP)
    pltpu.sync_copy(ann_hbm.at[pl.ds(ann0 + 0 * _NP, _NP)], bx1_v)
    pltpu.sync_copy(ann_hbm.at[pl.ds(ann0 + 1 * _NP, _NP)], by1_v)
    pltpu.sync_copy(ann_hbm.at[pl.ds(ann0 + 2 * _NP, _NP)], bx2_v)
    pltpu.sync_copy(ann_hbm.at[pl.ds(ann0 + 3 * _NP, _NP)], by2_v)
    pltpu.sync_copy(ann_hbm.at[pl.ds(ann0 + 4 * _NP, _NP)], bar_v)
    pltpu.sync_copy(ann_hbm.at[pl.ds(ann0 + 5 * _NP, _NP)], bcl_v)

    iota = lax.iota(jnp.int32, 16)
    iota4 = iota * 4
    iota80 = iota * _NUM_CLS
    ones = jnp.ones((16,), jnp.float32)
    zeros16 = jnp.zeros((16,), jnp.float32)
    zeros16i = jnp.zeros((16,), jnp.int32)
    ones16i = jnp.ones((16,), jnp.int32)

    def chunk_body(c, carry):
        a0 = q * _STRIDE + c * _CHUNK
        pltpu.sync_copy(anct_hbm.at[pl.ds(0 * _A + a0, _CHUNK)], ax1_v)
        pltpu.sync_copy(anct_hbm.at[pl.ds(1 * _A + a0, _CHUNK)], ay1_v)
        pltpu.sync_copy(anct_hbm.at[pl.ds(2 * _A + a0, _CHUNK)], ax2_v)
        pltpu.sync_copy(anct_hbm.at[pl.ds(3 * _A + a0, _CHUNK)], ay2_v)

        def epilogue(g, ax1, ay1, ax2, ay2, aw, ah, bi, bS, bj):
            s = pl.ds(g * 16, 16)
            ua = bS - bi
            pos = bi >= _POS_T * ua
            ign = jnp.logical_and(bi > _NEG_T * ua, jnp.logical_not(pos))
            st_v[s] = jnp.where(pos, 1.0, jnp.where(ign, -1.0, 0.0))

            gx1 = plsc.load_gather(bx1_v, [bj])
            gy1 = plsc.load_gather(by1_v, [bj])
            gx2 = plsc.load_gather(bx2_v, [bj])
            gy2 = plsc.load_gather(by2_v, [bj])
            gcl = plsc.load_gather(bcl_v, [bj])

            s5w = 5.0 / aw
            s5h = 5.0 / ah
            i4 = iota4 + g * 64
            plsc.store_scatter(reg_v, [i4], (gx1 - ax1) * s5w)
            plsc.store_scatter(reg_v, [i4 + 1], (gy1 - ay1) * s5h)
            plsc.store_scatter(reg_v, [i4 + 2], (gx2 - ax2) * s5w)
            plsc.store_scatter(reg_v, [i4 + 3], (gy2 - ay2) * s5h)

            base = g * (16 * _NUM_CLS)
            for k in range(_NUM_CLS):
                cls_v[pl.ds(base + k * 16, 16)] = zeros16
            label = gcl.astype(jnp.int32)
            ci = iota80 + (base + label)
            plsc.store_scatter(cls_v, [ci], ones, mask=pos)

        def group_body(g2, carry2):
            gA = g2 * 2
            gB = gA + 1
            sA = pl.ds(gA * 16, 16)
            sB = pl.ds(gB * 16, 16)
            ax1A = ax1_v[sA]; ay1A = ay1_v[sA]; ax2A = ax2_v[sA]; ay2A = ay2_v[sA]
            ax1B = ax1_v[sB]; ay1B = ay1_v[sB]; ax2B = ax2_v[sB]; ay2B = ay2_v[sB]
            awA = ax2A - ax1A; ahA = ay2A - ay1A; aareaA = awA * ahA
            awB = ax2B - ax1B; ahB = ay2B - ay1B; aareaB = awB * ahB

            def box_body(j, st8):
                biA, bSA, bjA, biB, bSB, bjB, jv = st8
                bx1 = plsc.load_gather(bx1_v, [jv])
                by1 = plsc.load_gather(by1_v, [jv])
                bx2 = plsc.load_gather(bx2_v, [jv])
                by2 = plsc.load_gather(by2_v, [jv])
                bar = plsc.load_gather(bar_v, [jv])
                iwA = jnp.maximum(jnp.minimum(ax2A, bx2) - jnp.maximum(ax1A, bx1), 0.0)
                ihA = jnp.maximum(jnp.minimum(ay2A, by2) - jnp.maximum(ay1A, by1), 0.0)
                iwB = jnp.maximum(jnp.minimum(ax2B, bx2) - jnp.maximum(ax1B, bx1), 0.0)
                ihB = jnp.maximum(jnp.minimum(ay2B, by2) - jnp.maximum(ay1B, by1), 0.0)
                interA = iwA * ihA
                interB = iwB * ihB
                SA = aareaA + bar
                SB = aareaB + bar
                mA = interA * bSA > biA * SA
                mB = interB * bSB > biB * SB
                biA = jnp.where(mA, interA, biA)
                bSA = jnp.where(mA, SA, bSA)
                bjA = jnp.where(mA, jv, bjA)
                biB = jnp.where(mB, interB, biB)
                bSB = jnp.where(mB, SB, bSB)
                bjB = jnp.where(mB, jv, bjB)
                return (biA, bSA, bjA, biB, bSB, bjB, jv + ones16i)

            biA, bSA, bjA, biB, bSB, bjB, _jv = lax.fori_loop(
                0, _N, box_body,
                (zeros16, ones, zeros16i, zeros16, ones, zeros16i, zeros16i))

            epilogue(gA, ax1A, ay1A, ax2A, ay2A, awA, ahA, biA, bSA, bjA)
            epilogue(gB, ax1B, ay1B, ax2B, ay2B, awB, ahB, biB, bSB, bjB)
            return carry2

        lax.fori_loop(0, _GROUPS // 2, group_body, None)

        row0 = b * _A + a0
        pltpu.sync_copy(cls_v, cls_hbm.at[pl.ds(row0 * _NUM_CLS, _CHUNK * _NUM_CLS)])
        pltpu.sync_copy(reg_v, reg_hbm.at[pl.ds(row0 * 4, _CHUNK * 4)])
        pltpu.sync_copy(st_v, st_hbm.at[pl.ds(row0, _CHUNK)])
        return carry

    lax.fori_loop(0, _NCHUNK, chunk_body, None)


def _sc_call(ann_flat, anct_flat):
    mesh = plsc.VectorSubcoreMesh(core_axis_name="c", subcore_axis_name="s")
    f = functools.partial(
        pl.kernel,
        out_type=[
            jax.ShapeDtypeStruct((_B * _A * _NUM_CLS,), jnp.float32),
            jax.ShapeDtypeStruct((_B * _A * 4,), jnp.float32),
            jax.ShapeDtypeStruct((_B * _A,), jnp.float32),
        ],
        mesh=mesh,
        compiler_params=pltpu.CompilerParams(needs_layout_passes=False),
        scratch_types=[
            pltpu.VMEM((_NP,), jnp.float32),
            pltpu.VMEM((_NP,), jnp.float32),
            pltpu.VMEM((_NP,), jnp.float32),
            pltpu.VMEM((_NP,), jnp.float32),
            pltpu.VMEM((_NP,), jnp.float32),
            pltpu.VMEM((_NP,), jnp.float32),
            pltpu.VMEM((_CHUNK,), jnp.float32),
            pltpu.VMEM((_CHUNK,), jnp.float32),
            pltpu.VMEM((_CHUNK,), jnp.float32),
            pltpu.VMEM((_CHUNK,), jnp.float32),
            pltpu.VMEM((_CHUNK * _NUM_CLS,), jnp.float32),
            pltpu.VMEM((_CHUNK * 4,), jnp.float32),
            pltpu.VMEM((_CHUNK,), jnp.float32),
        ],
    )(_sc_body)
    return f(ann_flat, anct_flat)


def kernel(annotations_batch, anchors):
    B, N, _ = annotations_batch.shape
    A = anchors.shape[0]
    assert (B, N, A) == (_B, _N, _A)
    boxes = annotations_batch[:, :, :4]
    barea = ((boxes[:, :, 2] - boxes[:, :, 0])
             * (boxes[:, :, 3] - boxes[:, :, 1]))[:, :, None]
    ann_rows = jnp.concatenate(
        [boxes, barea, annotations_batch[:, :, 4:5]], axis=-1)
    ann_t = jnp.transpose(ann_rows, (0, 2, 1))
    ann_t = jnp.pad(ann_t, ((0, 0), (0, 0), (0, _NP - _N)))
    ann_flat = ann_t.reshape(-1)
    anct_flat = jnp.transpose(anchors, (1, 0)).reshape(-1)
    cls, reg, st = _sc_call(ann_flat, anct_flat)
    return (cls.reshape(B, A, _NUM_CLS), reg.reshape(B, A, 4), st.reshape(B, A))
